# baseline (speedup 1.0000x reference)
import functools

import jax
import jax.numpy as jnp
from jax import lax
from jax.experimental import pallas as pl
from jax.experimental.pallas import tpu as pltpu
from jax.experimental.pallas import tpu_sc as plsc

VOCAB = 1000000
B = 16384
L = 200
N = B * L

_NC, _NS = 2, 16
_NW = _NC * _NS

_RSUB = 31264
_RLAST = VOCAB - (_NW - 1) * _RSUB
_NPIECE = 2
_PIECE = _RSUB // _NPIECE
_PLAST = _RLAST // _NPIECE

_PER_W = N // _NW
_CHUNK = 12800
_NCHUNK = _PER_W // _CHUNK

_mesh = functools.partial(
    plsc.VectorSubcoreMesh, core_axis_name="c", subcore_axis_name="s"
)


_TROWS = 8192
_TGRID = 123
_TPAD = _TGRID * _TROWS
_WROWS = _TROWS // 128
_WTOT = (3 * VOCAB + 383) // 384


def _transform_body(w_ref, sel_ref, tab_ref, out_ref):
    x = tab_ref[...]
    e = jax.lax.dot(x, sel_ref[...], precision=jax.lax.Precision.HIGHEST)
    e0 = e[:, 0:128]
    e1 = e[:, 128:256]
    e2 = e[:, 256:384]
    h0 = jnp.maximum(
        e0 * w_ref[0] + e1 * w_ref[1] + e2 * w_ref[2] + w_ref[3], 0.0)
    h1 = jnp.maximum(
        e0 * w_ref[4] + e1 * w_ref[5] + e2 * w_ref[6] + w_ref[7], 0.0)
    y = jnp.maximum(h0 * w_ref[8] + h1 * w_ref[9] + w_ref[10], 0.0)
    out_ref[...] = y


def _gather_body(t_hbm, idx_hbm, out_hbm, idx_v, g_v, sem):
    wid = lax.axis_index("s") * _NC + lax.axis_index("c")
    base = wid * _PER_W

    def gbody(k, carry):
        off = base + k * _CHUNK
        pltpu.sync_copy(idx_hbm.at[pl.ds(off, _CHUNK)], idx_v)
        pltpu.async_copy(t_hbm.at[idx_v], g_v, sem).wait()
        pltpu.sync_copy(g_v, out_hbm.at[pl.ds(off, _CHUNK)])
        return carry

    lax.fori_loop(0, _NCHUNK, gbody, 0)


def kernel(inputs, table, W1, b1, W2, b2):
    wvec = jnp.concatenate(
        [W1[0], b1[0:1], W1[1], b1[1:2], W2[0], b2]
    ).astype(jnp.float32)
    k = jnp.arange(384)
    sel = jax.nn.one_hot(128 * (k % 3) + k // 3, 384, dtype=jnp.float32)
    tab384 = jnp.pad(table.reshape(3 * VOCAB), (0, 384 * _WTOT - 3 * VOCAB))
    tab384 = tab384.reshape(_WTOT, 384)
    t2 = pl.pallas_call(
        _transform_body,
        grid=(_TGRID,),
        in_specs=[
            pl.BlockSpec(memory_space=pltpu.SMEM),
            pl.BlockSpec((384, 384), lambda i: (0, 0)),
            pl.BlockSpec((_WROWS, 384), lambda i: (i, 0)),
        ],
        out_specs=pl.BlockSpec((_WROWS, 128), lambda i: (i, 0)),
        out_shape=jax.ShapeDtypeStruct((_TGRID * _WROWS, 128), jnp.float32),
    )(wvec, sel, tab384)
    t = t2.reshape(_TPAD)

    gather = functools.partial(
        pl.kernel,
        mesh=_mesh(),
        out_type=jax.ShapeDtypeStruct((N,), jnp.float32),
        scratch_types=[
            pltpu.VMEM((_CHUNK,), jnp.int32),
            pltpu.VMEM((_CHUNK,), jnp.float32),
            pltpu.SemaphoreType.DMA,
        ],
    )(_gather_body)
    out = gather(t, inputs.reshape(N))
    return out.reshape(B, L, 1)


# speedup vs baseline: 14.4984x; 14.4984x over previous
import functools

import jax
import jax.numpy as jnp
from jax import lax
from jax.experimental import pallas as pl
from jax.experimental.pallas import tpu as pltpu
from jax.experimental.pallas import tpu_sc as plsc

VOCAB = 1000000
B = 16384
L = 200
N = B * L

_NC, _NS = 2, 16
_NW = _NC * _NS

_RSUB = 31264
_RLAST = VOCAB - (_NW - 1) * _RSUB
_NPIECE = 2
_PIECE = _RSUB // _NPIECE
_PLAST = _RLAST // _NPIECE

_PER_W = N // _NW
_CHUNK = 12800
_NCHUNK = _PER_W // _CHUNK

_mesh = functools.partial(
    plsc.VectorSubcoreMesh, core_axis_name="c", subcore_axis_name="s"
)


_TCOLS = 65536
_TGRID = 16
_TPAD = _TGRID * _TCOLS


def _transform_body(w_ref, tab_ref, out_ref):
    x = tab_ref[...]
    e0 = x[0:1, :]
    e1 = x[1:2, :]
    e2 = x[2:3, :]
    h0 = jnp.maximum(
        e0 * w_ref[0] + e1 * w_ref[1] + e2 * w_ref[2] + w_ref[3], 0.0)
    h1 = jnp.maximum(
        e0 * w_ref[4] + e1 * w_ref[5] + e2 * w_ref[6] + w_ref[7], 0.0)
    y = jnp.maximum(h0 * w_ref[8] + h1 * w_ref[9] + w_ref[10], 0.0)
    out_ref[...] = y.reshape(_TCOLS // 128, 128)


def _gather_body(t_hbm, idx_hbm, out_hbm, idx_v, g_v, sem):
    wid = lax.axis_index("s") * _NC + lax.axis_index("c")
    base = wid * _PER_W

    def gbody(k, carry):
        off = base + k * _CHUNK
        pltpu.sync_copy(idx_hbm.at[pl.ds(off, _CHUNK)], idx_v)
        pltpu.async_copy(t_hbm.at[idx_v], g_v, sem).wait()
        pltpu.sync_copy(g_v, out_hbm.at[pl.ds(off, _CHUNK)])
        return carry

    lax.fori_loop(0, _NCHUNK, gbody, 0)


def kernel(inputs, table, W1, b1, W2, b2):
    wvec = jnp.concatenate(
        [W1[0], b1[0:1], W1[1], b1[1:2], W2[0], b2]
    ).astype(jnp.float32)
    t2 = pl.pallas_call(
        _transform_body,
        grid=(_TGRID,),
        in_specs=[
            pl.BlockSpec(memory_space=pltpu.SMEM),
            pl.BlockSpec((3, _TCOLS), lambda i: (0, i)),
        ],
        out_specs=pl.BlockSpec((_TCOLS // 128, 128), lambda i: (i, 0)),
        out_shape=jax.ShapeDtypeStruct((_TPAD // 128, 128), jnp.float32),
    )(wvec, table.T)
    t = t2.reshape(_TPAD)

    gather = functools.partial(
        pl.kernel,
        mesh=_mesh(),
        out_type=jax.ShapeDtypeStruct((N,), jnp.float32),
        scratch_types=[
            pltpu.VMEM((_CHUNK,), jnp.int32),
            pltpu.VMEM((_CHUNK,), jnp.float32),
            pltpu.SemaphoreType.DMA,
        ],
    )(_gather_body)
    out = gather(t, inputs.reshape(N))
    return out.reshape(B, L, 1)


# speedup vs baseline: 14.8101x; 1.0215x over previous
import functools

import jax
import jax.numpy as jnp
from jax import lax
from jax.experimental import pallas as pl
from jax.experimental.pallas import tpu as pltpu
from jax.experimental.pallas import tpu_sc as plsc

VOCAB = 1000000
B = 16384
L = 200
N = B * L

_NC, _NS = 2, 16
_NW = _NC * _NS

_RSUB = 31264
_RLAST = VOCAB - (_NW - 1) * _RSUB
_NPIECE = 2
_PIECE = _RSUB // _NPIECE
_PLAST = _RLAST // _NPIECE

_PER_W = N // _NW
_CHUNK = 12800
_NCHUNK = _PER_W // _CHUNK

_mesh = functools.partial(
    plsc.VectorSubcoreMesh, core_axis_name="c", subcore_axis_name="s"
)


_TCOLS = 65536
_TGRID = 16
_TPAD = _TGRID * _TCOLS


def _transform_body(w_ref, tab_ref, out_ref):
    x = tab_ref[...]
    e0 = x[0:1, :]
    e1 = x[1:2, :]
    e2 = x[2:3, :]
    h0 = jnp.maximum(
        e0 * w_ref[0] + e1 * w_ref[1] + e2 * w_ref[2] + w_ref[3], 0.0)
    h1 = jnp.maximum(
        e0 * w_ref[4] + e1 * w_ref[5] + e2 * w_ref[6] + w_ref[7], 0.0)
    y = jnp.maximum(h0 * w_ref[8] + h1 * w_ref[9] + w_ref[10], 0.0)
    out_ref[...] = y.reshape(_TCOLS // 128, 128)


def _gather_body(t_hbm, idx_hbm, out_hbm, ia, ib, ga, gb,
                 sia, sib, sga, sgb, soa, sob):
    wid = lax.axis_index("s") * _NC + lax.axis_index("c")
    base = wid * _PER_W
    iv, gv = [ia, ib], [ga, gb]
    si, sg, so = [sia, sib], [sga, sgb], [soa, sob]

    def islice(k):
        return idx_hbm.at[pl.ds(base + k * _CHUNK, _CHUNK)]

    def oslice(k):
        return out_hbm.at[pl.ds(base + k * _CHUNK, _CHUNK)]

    hi = [None] * _NCHUNK
    hg = [None] * _NCHUNK
    ho = [None] * _NCHUNK
    hi[0] = pltpu.async_copy(islice(0), iv[0], si[0])
    for k in range(_NCHUNK):
        b = k % 2
        o = 1 - b
        if k >= 2:
            ho[k - 2].wait()
        hi[k].wait()
        hg[k] = pltpu.async_copy(t_hbm.at[iv[b]], gv[b], sg[b])
        if k + 1 < _NCHUNK:
            hi[k + 1] = pltpu.async_copy(islice(k + 1), iv[o], si[o])
        hg[k].wait()
        ho[k] = pltpu.async_copy(gv[b], oslice(k), so[b])
    ho[_NCHUNK - 2].wait()
    ho[_NCHUNK - 1].wait()


def kernel(inputs, table, W1, b1, W2, b2):
    wvec = jnp.concatenate(
        [W1[0], b1[0:1], W1[1], b1[1:2], W2[0], b2]
    ).astype(jnp.float32)
    t2 = pl.pallas_call(
        _transform_body,
        grid=(_TGRID,),
        in_specs=[
            pl.BlockSpec(memory_space=pltpu.SMEM),
            pl.BlockSpec((3, _TCOLS), lambda i: (0, i)),
        ],
        out_specs=pl.BlockSpec((_TCOLS // 128, 128), lambda i: (i, 0)),
        out_shape=jax.ShapeDtypeStruct((_TPAD // 128, 128), jnp.float32),
    )(wvec, table.T)
    t = t2.reshape(_TPAD)

    gather = functools.partial(
        pl.kernel,
        mesh=_mesh(),
        out_type=jax.ShapeDtypeStruct((N,), jnp.float32),
        scratch_types=[
            pltpu.VMEM((_CHUNK,), jnp.int32),
            pltpu.VMEM((_CHUNK,), jnp.int32),
            pltpu.VMEM((_CHUNK,), jnp.float32),
            pltpu.VMEM((_CHUNK,), jnp.float32),
            pltpu.SemaphoreType.DMA,
            pltpu.SemaphoreType.DMA,
            pltpu.SemaphoreType.DMA,
            pltpu.SemaphoreType.DMA,
            pltpu.SemaphoreType.DMA,
            pltpu.SemaphoreType.DMA,
        ],
    )(_gather_body)
    out = gather(t, inputs.reshape(N))
    return out.reshape(B, L, 1)


# speedup vs baseline: 14.8212x; 1.0007x over previous
import functools

import jax
import jax.numpy as jnp
from jax import lax
from jax.experimental import pallas as pl
from jax.experimental.pallas import tpu as pltpu
from jax.experimental.pallas import tpu_sc as plsc

VOCAB = 1000000
B = 16384
L = 200
N = B * L

_NC, _NS = 2, 16
_NW = _NC * _NS

_RSUB = 31264
_RLAST = VOCAB - (_NW - 1) * _RSUB
_NPIECE = 2
_PIECE = _RSUB // _NPIECE
_PLAST = _RLAST // _NPIECE

_PER_W = N // _NW
_CHUNK = 12800
_NCHUNK = _PER_W // _CHUNK

_mesh = functools.partial(
    plsc.VectorSubcoreMesh, core_axis_name="c", subcore_axis_name="s"
)


_TCOLS = 65536
_TGRID = 16
_TPAD = _TGRID * _TCOLS


def _transform_body(w_ref, tab_ref, out_ref):
    x = tab_ref[...]
    e0 = x[0:1, :]
    e1 = x[1:2, :]
    e2 = x[2:3, :]
    h0 = jnp.maximum(
        e0 * w_ref[0] + e1 * w_ref[1] + e2 * w_ref[2] + w_ref[3], 0.0)
    h1 = jnp.maximum(
        e0 * w_ref[4] + e1 * w_ref[5] + e2 * w_ref[6] + w_ref[7], 0.0)
    y = jnp.maximum(h0 * w_ref[8] + h1 * w_ref[9] + w_ref[10], 0.0)
    out_ref[...] = y.reshape(_TCOLS)


def _gather_body(t_hbm, idx_hbm, out_hbm, ia, ib, ga, gb,
                 sia, sib, sga, sgb, soa, sob):
    wid = lax.axis_index("s") * _NC + lax.axis_index("c")
    base = wid * _PER_W
    iv, gv = [ia, ib], [ga, gb]
    si, sg, so = [sia, sib], [sga, sgb], [soa, sob]

    def islice(k):
        return idx_hbm.at[pl.ds(base + k * _CHUNK, _CHUNK)]

    def oslice(k):
        return out_hbm.at[pl.ds(base + k * _CHUNK, _CHUNK)]

    hi = [None] * _NCHUNK
    hg = [None] * _NCHUNK
    ho = [None] * _NCHUNK
    hi[0] = pltpu.async_copy(islice(0), iv[0], si[0])
    for k in range(_NCHUNK):
        b = k % 2
        o = 1 - b
        if k >= 2:
            ho[k - 2].wait()
        hi[k].wait()
        hg[k] = pltpu.async_copy(t_hbm.at[iv[b]], gv[b], sg[b])
        if k + 1 < _NCHUNK:
            hi[k + 1] = pltpu.async_copy(islice(k + 1), iv[o], si[o])
        hg[k].wait()
        ho[k] = pltpu.async_copy(gv[b], oslice(k), so[b])
    ho[_NCHUNK - 2].wait()
    ho[_NCHUNK - 1].wait()


def kernel(inputs, table, W1, b1, W2, b2):
    wvec = jnp.concatenate(
        [W1[0], b1[0:1], W1[1], b1[1:2], W2[0], b2]
    ).astype(jnp.float32)
    t2 = pl.pallas_call(
        _transform_body,
        grid=(_TGRID,),
        in_specs=[
            pl.BlockSpec(memory_space=pltpu.SMEM),
            pl.BlockSpec((3, _TCOLS), lambda i: (0, i)),
        ],
        out_specs=pl.BlockSpec((_TCOLS,), lambda i: (i,)),
        out_shape=jax.ShapeDtypeStruct((_TPAD,), jnp.float32),
    )(wvec, table.T)
    t = t2

    gather = functools.partial(
        pl.kernel,
        mesh=_mesh(),
        out_type=jax.ShapeDtypeStruct((N,), jnp.float32),
        scratch_types=[
            pltpu.VMEM((_CHUNK,), jnp.int32),
            pltpu.VMEM((_CHUNK,), jnp.int32),
            pltpu.VMEM((_CHUNK,), jnp.float32),
            pltpu.VMEM((_CHUNK,), jnp.float32),
            pltpu.SemaphoreType.DMA,
            pltpu.SemaphoreType.DMA,
            pltpu.SemaphoreType.DMA,
            pltpu.SemaphoreType.DMA,
            pltpu.SemaphoreType.DMA,
            pltpu.SemaphoreType.DMA,
        ],
    )(_gather_body)
    out = gather(t, inputs.reshape(N))
    return out.reshape(B, L, 1)
